# Initial kernel scaffold; baseline (speedup 1.0000x reference)
import jax
import jax.numpy as jnp
from jax.experimental import pallas as pl


def kernel(features, edge_index, Ws, Wn, b):
    raise NotImplementedError("write your pallas kernel here")


# speedup vs baseline: 5.2392x; 5.2392x over previous
import functools

import jax
import jax.numpy as jnp
from jax import lax
from jax.experimental import pallas as pl
from jax.experimental.pallas import tpu as pltpu
from jax.experimental.pallas import tpu_sc as plsc

C = 50
G = 8
NC = 2
NS = 16
NW = NC * NS
SB = 80


def _sc_pass(N, D, E, mode):
    NCH = E // (C * NW)
    NG = NCH // G
    NPAD = -(-N // (NS * SB)) * (NS * SB)
    NZ = NPAD // (NS * SB)
    NO = N // SB
    NOT = -(-NO // NS)

    mesh = plsc.VectorSubcoreMesh(core_axis_name="c", subcore_axis_name="s")

    scratch = [
        pltpu.VMEM((G, C), jnp.int32),
        pltpu.VMEM((C, D), jnp.float32),
        pltpu.VMEM((SB, D), jnp.float32),
        pltpu.VMEM_SHARED((NPAD, D), jnp.float32),
    ]
    if mode == "agg":
        scratch.insert(0, pltpu.VMEM((G, C), jnp.int32))

    @functools.partial(pl.kernel, mesh=mesh,
                       out_type=jax.ShapeDtypeStruct((NC * N, D),
                                                     jnp.float32),
                       scratch_types=scratch)
    def body(*refs):
        if mode == "agg":
            (h_hbm, src_hbm, dst_hbm, zrows_hbm,
             out_hbm, idxs, idxd, rows, zbuf, acc_sh) = refs
        else:
            (dst_hbm, zrows_hbm, ones_hbm,
             out_hbm, idxd, rows, zbuf, acc_sh) = refs
        cid = lax.axis_index("c")
        sid = lax.axis_index("s")
        w = cid * NS + sid

        pltpu.sync_copy(zrows_hbm, zbuf)
        for t in range(NZ):
            z0 = (sid * NZ + t) * SB
            pltpu.sync_copy(zbuf, acc_sh.at[pl.ds(z0, SB)])
        if mode == "deg":
            pltpu.sync_copy(ones_hbm, rows)
        plsc.subcore_barrier()

        @pl.loop(0, NG)
        def _(g):
            base = w * NCH + g * G
            pltpu.sync_copy(dst_hbm.at[pl.ds(base, G)], idxd)
            if mode == "agg":
                pltpu.sync_copy(src_hbm.at[pl.ds(base, G)], idxs)
            for k in range(G):
                if mode == "agg":
                    pltpu.sync_copy(h_hbm.at[idxs.at[k]], rows)
                pltpu.sync_copy(rows, acc_sh.at[idxd.at[k]],
                                add=True)

        plsc.subcore_barrier()
        for t in range(NOT):
            c = sid + t * NS

            @pl.when(c < NO)
            def _():
                r0 = c * SB
                pltpu.sync_copy(acc_sh.at[pl.ds(r0, SB)], zbuf)
                pltpu.sync_copy(zbuf, out_hbm.at[pl.ds(cid * N + r0, SB)])

    return body


def _tc_layer(N, D, relu):
    BM = 1000
    nblk = N // BM

    def body(h_ref, a0_ref, a1_ref, d0_ref, d1_ref, ws_ref, wn_ref, b_ref,
             o_ref):
        deg = jnp.maximum(d0_ref[:, :1] + d1_ref[:, :1], 1.0)
        mean = (a0_ref[:, :] + a1_ref[:, :]) / deg
        acc = jnp.dot(h_ref[:, :], ws_ref[:, :],
                      preferred_element_type=jnp.float32)
        acc = acc + jnp.dot(mean, wn_ref[:, :],
                            preferred_element_type=jnp.float32)
        acc = acc + b_ref[:, :]
        o_ref[:, :] = jnp.maximum(acc, 0.0) if relu else acc

    return pl.pallas_call(
        body,
        grid=(nblk,),
        in_specs=[
            pl.BlockSpec((BM, D), lambda i: (i, 0)),
            pl.BlockSpec((BM, D), lambda i: (i, 0)),
            pl.BlockSpec((BM, D), lambda i: (i + nblk, 0)),
            pl.BlockSpec((BM, D), lambda i: (i, 0)),
            pl.BlockSpec((BM, D), lambda i: (i + nblk, 0)),
            pl.BlockSpec((D, D), lambda i: (0, 0)),
            pl.BlockSpec((D, D), lambda i: (0, 0)),
            pl.BlockSpec((1, D), lambda i: (0, 0)),
        ],
        out_specs=pl.BlockSpec((BM, D), lambda i: (i, 0)),
        out_shape=jax.ShapeDtypeStruct((N, D), jnp.float32),
    )


def kernel(features, edge_index, Ws, Wn, b):
    N, D = features.shape
    E = edge_index.shape[1]
    L = Ws.shape[0]

    src2d = edge_index[0].reshape(-1, C)
    dst2d = edge_index[1].reshape(-1, C)
    zrows = jnp.zeros((SB, D), jnp.float32)
    ones = jnp.ones((C, D), jnp.float32)

    deg_fn = _sc_pass(N, D, E, mode="deg")
    agg_fn = _sc_pass(N, D, E, mode="agg")
    deg = deg_fn(dst2d, zrows, ones)

    h = features
    for i in range(L):
        agg = agg_fn(h, src2d, dst2d, zrows)
        h = _tc_layer(N, D, relu=(i < L - 1))(
            h, agg, agg, deg, deg, Ws[i], Wn[i], b[i].reshape(1, D))
    return h


# speedup vs baseline: 9.0907x; 1.7351x over previous
import functools

import jax
import jax.numpy as jnp
from jax import lax
from jax.experimental import pallas as pl
from jax.experimental.pallas import tpu as pltpu
from jax.experimental.pallas import tpu_sc as plsc

C = 125
G = 8
NC = 2
NS = 16
NW = NC * NS
SB = 80


def _sc_pass(N, D, E, mode):
    NCH = E // (C * NW)
    NG = NCH // G
    NPAD = -(-N // (NS * SB)) * (NS * SB)
    NZ = NPAD // (NS * SB)
    NO = N // SB
    NOT = -(-NO // NS)

    mesh = plsc.VectorSubcoreMesh(core_axis_name="c", subcore_axis_name="s")

    scratch = [
        pltpu.VMEM((G, C), jnp.int32),
        pltpu.VMEM((C, D), jnp.float32),
        pltpu.VMEM((SB, D), jnp.float32),
        pltpu.VMEM_SHARED((NPAD, D), jnp.float32),
    ]
    if mode == "agg":
        scratch = ([pltpu.VMEM((G, C), jnp.int32)] + scratch
                   + [pltpu.VMEM((C, D), jnp.float32),
                      pltpu.SemaphoreType.DMA,
                      pltpu.SemaphoreType.DMA])

    @functools.partial(pl.kernel, mesh=mesh,
                       out_type=jax.ShapeDtypeStruct((NC * N, D),
                                                     jnp.float32),
                       scratch_types=scratch)
    def body(*refs):
        if mode == "agg":
            (h_hbm, src_hbm, dst_hbm, zrows_hbm,
             out_hbm, idxs, idxd, rows_a, zbuf, acc_sh,
             rows_b, sem_a, sem_b) = refs
            rows = (rows_a, rows_b)
            sems = (sem_a, sem_b)
        else:
            (dst_hbm, zrows_hbm, ones_hbm,
             out_hbm, idxd, ones_v, zbuf, acc_sh) = refs
        cid = lax.axis_index("c")
        sid = lax.axis_index("s")
        w = cid * NS + sid

        pltpu.sync_copy(zrows_hbm, zbuf)
        for t in range(NZ):
            z0 = (sid * NZ + t) * SB
            pltpu.sync_copy(zbuf, acc_sh.at[pl.ds(z0, SB)])
        if mode == "deg":
            pltpu.sync_copy(ones_hbm, ones_v)
        plsc.subcore_barrier()

        if mode == "agg":

            @pl.loop(0, NG)
            def _(g):
                base = w * NCH + g * G
                pltpu.sync_copy(src_hbm.at[pl.ds(base, G)], idxs)
                pltpu.sync_copy(dst_hbm.at[pl.ds(base, G)], idxd)
                pltpu.async_copy(h_hbm.at[idxs.at[0]], rows[0], sems[0])
                for k in range(G):
                    cur, csem = rows[k % 2], sems[k % 2]
                    pltpu.make_async_copy(
                        h_hbm.at[idxs.at[k]], cur, csem).wait()
                    if k + 1 < G:
                        pltpu.async_copy(h_hbm.at[idxs.at[k + 1]],
                                         rows[(k + 1) % 2],
                                         sems[(k + 1) % 2])
                    pltpu.sync_copy(cur, acc_sh.at[idxd.at[k]], add=True)
        else:

            @pl.loop(0, NG)
            def _(g):
                base = w * NCH + g * G
                pltpu.sync_copy(dst_hbm.at[pl.ds(base, G)], idxd)
                for k in range(G):
                    pltpu.sync_copy(ones_v, acc_sh.at[idxd.at[k]], add=True)

        plsc.subcore_barrier()
        for t in range(NOT):
            c = sid + t * NS

            @pl.when(c < NO)
            def _():
                r0 = c * SB
                pltpu.sync_copy(acc_sh.at[pl.ds(r0, SB)], zbuf)
                pltpu.sync_copy(zbuf, out_hbm.at[pl.ds(cid * N + r0, SB)])

    return body


def _tc_layer(N, D, relu):
    BM = 1000
    nblk = N // BM

    def body(h_ref, a0_ref, a1_ref, d0_ref, d1_ref, ws_ref, wn_ref, b_ref,
             o_ref):
        deg = jnp.maximum(d0_ref[:, :1] + d1_ref[:, :1], 1.0)
        mean = (a0_ref[:, :] + a1_ref[:, :]) / deg
        acc = jnp.dot(h_ref[:, :], ws_ref[:, :],
                      preferred_element_type=jnp.float32)
        acc = acc + jnp.dot(mean, wn_ref[:, :],
                            preferred_element_type=jnp.float32)
        acc = acc + b_ref[:, :]
        o_ref[:, :] = jnp.maximum(acc, 0.0) if relu else acc

    return pl.pallas_call(
        body,
        grid=(nblk,),
        in_specs=[
            pl.BlockSpec((BM, D), lambda i: (i, 0)),
            pl.BlockSpec((BM, D), lambda i: (i, 0)),
            pl.BlockSpec((BM, D), lambda i: (i + nblk, 0)),
            pl.BlockSpec((BM, D), lambda i: (i, 0)),
            pl.BlockSpec((BM, D), lambda i: (i + nblk, 0)),
            pl.BlockSpec((D, D), lambda i: (0, 0)),
            pl.BlockSpec((D, D), lambda i: (0, 0)),
            pl.BlockSpec((1, D), lambda i: (0, 0)),
        ],
        out_specs=pl.BlockSpec((BM, D), lambda i: (i, 0)),
        out_shape=jax.ShapeDtypeStruct((N, D), jnp.float32),
    )


def kernel(features, edge_index, Ws, Wn, b):
    N, D = features.shape
    E = edge_index.shape[1]
    L = Ws.shape[0]

    src2d = edge_index[0].reshape(-1, C)
    dst2d = edge_index[1].reshape(-1, C)
    zrows = jnp.zeros((SB, D), jnp.float32)
    ones = jnp.ones((C, D), jnp.float32)

    deg_fn = _sc_pass(N, D, E, mode="deg")
    agg_fn = _sc_pass(N, D, E, mode="agg")
    deg = deg_fn(dst2d, zrows, ones)

    h = features
    for i in range(L):
        agg = agg_fn(h, src2d, dst2d, zrows)
        h = _tc_layer(N, D, relu=(i < L - 1))(
            h, agg, agg, deg, deg, Ws[i], Wn[i], b[i].reshape(1, D))
    return h
